# baseline (speedup 1.0000x reference)
import functools
import jax
import jax.numpy as jnp
from jax import lax
from jax.experimental import pallas as pl
from jax.experimental.pallas import tpu as pltpu
from jax.experimental.pallas import tpu_sc as plsc

_NUM_CORES = 2
_NUM_SUBCORES = 16
_NW = _NUM_CORES * _NUM_SUBCORES
_LANES = 16

_D = 64
_NQ = _D // _LANES
_SCALE = 8.0
_BB = 128


def _make_gather(n_t: int, n_b: int):
    assert n_b == _NW * _BB
    mesh = plsc.VectorSubcoreMesh(core_axis_name="c", subcore_axis_name="s")

    @functools.partial(
        pl.kernel,
        out_type=jax.ShapeDtypeStruct((n_t, _D // 8, n_b // _BB, 8, _BB),
                                      jnp.float32),
        mesh=mesh,
        scratch_types=(
            [pltpu.VMEM((_BB,), jnp.int32)] * 2
            + [pltpu.VMEM((_BB, _LANES), jnp.float32)] * (2 * _NQ)
            + [pltpu.VMEM((_D // 8, 8, _BB), jnp.float32)] * 2
            + [pltpu.SemaphoreType.DMA] * 4
        ),
        compiler_params=pltpu.CompilerParams(use_tc_tiling_on_sc=False,
                                             needs_layout_passes=False),
    )
    def gather_scale(ids_hbm, t0_hbm, t1_hbm, t2_hbm, t3_hbm, out_hbm,
                     idx0, idx1, r00, r01, r02, r03, r10, r11, r12, r13,
                     trows0, trows1, sem0, sem1, wsem0, wsem1):
        tq_hbm = (t0_hbm, t1_hbm, t2_hbm, t3_hbm)
        rows_sets = ((r00, r01, r02, r03), (r10, r11, r12, r13))
        wid = lax.axis_index("s") * _NUM_CORES + lax.axis_index("c")

        def row_copies(idx_v, rows, sem):
            return [
                pltpu.make_async_copy(tq_hbm[q].at[idx_v], rows[q], sem)
                for q in range(_NQ)
            ]

        def fetch(t, idx_v, rows, sem):
            pltpu.sync_copy(ids_hbm.at[t, wid], idx_v)
            for c in row_copies(idx_v, rows, sem):
                c.start()

        def out_copies(trows, t, wsem):
            return [
                pltpu.make_async_copy(trows.at[a], out_hbm.at[t, a, wid],
                                      wsem)
                for a in range(_D // 8)
            ]

        def process_block(rows, t, trows, wsem):
            @pl.when(t >= 2)
            def _():
                for c in out_copies(trows, t, wsem):
                    c.wait()

            iota = lax.iota(jnp.int32, _LANES)
            row_vecs = tuple(iota + _LANES * j
                             for j in range(_BB // _LANES))

            @plsc.parallel_loop(0, _LANES, unroll=2)
            def k_body(k):
                rot = lax.rem(iota + k, _LANES)
                a_base = rot >> 3
                r_vec = rot & 7
                for m in range(_NQ):
                    a_vec = a_base + 2 * m
                    for j in range(_BB // _LANES):
                        v = plsc.load_gather(rows[m], [row_vecs[j], rot])
                        plsc.store_scatter(trows, [a_vec, r_vec, row_vecs[j]],
                                           v * _SCALE)

            for c in out_copies(trows, t, wsem):
                c.start()

        fetch(0, idx0, rows_sets[0], sem0)

        def t_body(k, _):
            t0 = 2 * k
            fetch(t0 + 1, idx1, rows_sets[1], sem1)
            for c in row_copies(idx0, rows_sets[0], sem0):
                c.wait()
            process_block(rows_sets[0], t0, trows0, wsem0)

            @pl.when(t0 + 2 < n_t)
            def _():
                fetch(t0 + 2, idx0, rows_sets[0], sem0)
            for c in row_copies(idx1, rows_sets[1], sem1):
                c.wait()
            process_block(rows_sets[1], t0 + 1, trows1, wsem1)
            return 0

        lax.fori_loop(0, n_t // 2, t_body, 0)
        for c in out_copies(trows0, n_t - 2, wsem0):
            c.wait()
        for c in out_copies(trows1, n_t - 1, wsem1):
            c.wait()

    return gather_scale


def kernel(input_ids, table):
    b, t = input_ids.shape
    ids3 = input_ids.T.reshape(t, b // _BB, _BB).astype(jnp.int32)
    quarters = [table[:, q * _LANES:(q + 1) * _LANES] for q in range(_NQ)]
    out_phys = _make_gather(t, b)(ids3, *quarters)
    out = out_phys.transpose(2, 4, 0, 1, 3).reshape(b, t, _D)
    return out


# speedup vs baseline: 2.9858x; 2.9858x over previous
import functools
import jax
import jax.numpy as jnp
from jax import lax
from jax.experimental import pallas as pl
from jax.experimental.pallas import tpu as pltpu
from jax.experimental.pallas import tpu_sc as plsc

_NUM_CORES = 2
_NUM_SUBCORES = 16
_NW = _NUM_CORES * _NUM_SUBCORES
_LANES = 16

_D = 64
_SCALE = 8.0
_BB = 128


def _make_gather(n_t: int, n_b: int):
    assert n_b == _NW * _BB
    mesh = plsc.VectorSubcoreMesh(core_axis_name="c", subcore_axis_name="s")

    @functools.partial(
        pl.kernel,
        out_type=jax.ShapeDtypeStruct((n_t, _D // 8, n_b // _BB, 8, _BB),
                                      jnp.float32),
        mesh=mesh,
        scratch_types=[
            pltpu.VMEM((_BB,), jnp.int32),
            pltpu.VMEM((_BB,), jnp.int32),
            pltpu.VMEM((_BB, _D), jnp.float32),
            pltpu.VMEM((_BB, _D), jnp.float32),
            pltpu.VMEM((_D // 8, 8, _BB), jnp.float32),
            pltpu.VMEM((_D // 8, 8, _BB), jnp.float32),
            pltpu.SemaphoreType.DMA,
            pltpu.SemaphoreType.DMA,
            pltpu.SemaphoreType.DMA,
            pltpu.SemaphoreType.DMA,
        ],
        compiler_params=pltpu.CompilerParams(use_tc_tiling_on_sc=False,
                                             needs_layout_passes=False),
    )
    def gather_scale(ids_hbm, table_hbm, out_hbm,
                     idx0, idx1, rows0, rows1, trows0, trows1,
                     sem0, sem1, wsem0, wsem1):
        wid = lax.axis_index("s") * _NUM_CORES + lax.axis_index("c")

        def fetch(t, idx_v, rows_v, sem):
            pltpu.sync_copy(ids_hbm.at[t, wid], idx_v)
            pltpu.make_async_copy(table_hbm.at[idx_v], rows_v, sem).start()

        def out_copies(trows, t, wsem):
            return [
                pltpu.make_async_copy(trows.at[a], out_hbm.at[t, a, wid],
                                      wsem)
                for a in range(_D // 8)
            ]

        def process_block(rows_v, t, trows, wsem):
            @pl.when(t >= 2)
            def _():
                for c in out_copies(trows, t, wsem):
                    c.wait()

            iota = lax.iota(jnp.int32, _LANES)
            row_vecs = tuple(iota + _LANES * j
                             for j in range(_BB // _LANES))

            @plsc.parallel_loop(0, _LANES, unroll=4)
            def k_body(k):
                rot = lax.rem(iota + k, _LANES)
                a_base = rot >> 3
                r_vec = rot & 7
                for m in range(_D // _LANES):
                    col = rot + (_LANES * m)
                    a_vec = a_base + 2 * m
                    for j in range(_BB // _LANES):
                        v = plsc.load_gather(rows_v, [row_vecs[j], col])
                        plsc.store_scatter(trows, [a_vec, r_vec, row_vecs[j]],
                                           v * _SCALE)

            for c in out_copies(trows, t, wsem):
                c.start()

        fetch(0, idx0, rows0, sem0)

        def t_body(k, _):
            t0 = 2 * k
            fetch(t0 + 1, idx1, rows1, sem1)
            pltpu.make_async_copy(table_hbm.at[idx0], rows0, sem0).wait()
            process_block(rows0, t0, trows0, wsem0)

            @pl.when(t0 + 2 < n_t)
            def _():
                fetch(t0 + 2, idx0, rows0, sem0)
            pltpu.make_async_copy(table_hbm.at[idx1], rows1, sem1).wait()
            process_block(rows1, t0 + 1, trows1, wsem1)
            return 0

        lax.fori_loop(0, n_t // 2, t_body, 0)
        for c in out_copies(trows0, n_t - 2, wsem0):
            c.wait()
        for c in out_copies(trows1, n_t - 1, wsem1):
            c.wait()

    return gather_scale


def kernel(input_ids, table):
    b, t = input_ids.shape
    ids3 = input_ids.T.reshape(t, b // _BB, _BB).astype(jnp.int32)
    out_phys = _make_gather(t, b)(ids3, table)
    out = out_phys.transpose(2, 4, 0, 1, 3).reshape(b, t, _D)
    return out
